# baseline (device time: 23049 ns/iter reference)
import jax
import jax.numpy as jnp
from jax import lax
from jax.experimental import pallas as pl
from jax.experimental.pallas import tpu as pltpu

N_DEV = 16
N_TOK = 512
N_EXP = 32
D_OUT = 512
CHUNK = N_TOK // N_DEV
QROWS = 4 * CHUNK


def kernel(x, router_W, route_idx, expert_W, shared_W):
    def body(x_ref, rw_ref, idx_ref, ew_ref, sw_ref, out_ref,
             acc_ref, send_ref, stage1, stage2, send_sems, recv_sems):
        p = lax.axis_index("i")
        z = p // 4
        q = p % 4
        qx = q ^ 1
        qy = 3 - q
        qd = 3 - qx

        plane_partners = [4 * z + qx, 4 * z + qy, 4 * z + qd]
        col_partners = [4 * (z ^ 1) + q, 4 * (z ^ 2) + q, 4 * (z ^ 3) + q]

        barrier_sem = pltpu.get_barrier_semaphore()
        for nbr in plane_partners + col_partners:
            pl.semaphore_signal(barrier_sem, inc=1, device_id=(nbr,),
                                device_id_type=pl.DeviceIdType.MESH)
        pl.semaphore_wait(barrier_sem, 6)

        e0 = 2 * p

        def masked_partial(chunks):
            xs = jnp.concatenate(
                [x_ref[pl.ds(c * CHUNK, CHUNK), :] for c in chunks], axis=0)
            idxs = jnp.concatenate(
                [idx_ref[pl.ds(c * CHUNK, CHUNK), :] for c in chunks], axis=0)
            n = CHUNK * len(chunks)
            scores = jnp.dot(xs, rw_ref[:, :],
                             preferred_element_type=jnp.float32)
            s_max = jnp.max(scores, axis=1, keepdims=True)
            ex = jnp.exp(scores - s_max)
            probs = ex / jnp.sum(ex, axis=1, keepdims=True)
            onehot = lax.broadcasted_iota(jnp.int32, (n, N_EXP), 1) == idxs
            p_sel = jnp.sum(jnp.where(onehot, probs, 0.0), axis=1,
                            keepdims=True)
            y0 = jnp.dot(xs, ew_ref[0], preferred_element_type=jnp.float32)
            y1 = jnp.dot(xs, ew_ref[1], preferred_element_type=jnp.float32)
            c0 = jnp.where(idxs == e0, p_sel, 0.0)
            c1 = jnp.where(idxs == e0 + 1, p_sel, 0.0)
            return c0 * y0 + c1 * y1

        p1_rdmas = []
        for j, qq in enumerate([qx, qy, qd]):
            rows = pl.ds(j * QROWS, QROWS)
            send_ref[rows, :] = masked_partial([4 * Z + qq for Z in range(4)])
            r = pltpu.make_async_remote_copy(
                src_ref=send_ref.at[rows, :],
                dst_ref=stage1.at[rows, :],
                send_sem=send_sems.at[j], recv_sem=recv_sems.at[j],
                device_id=(plane_partners[j],),
                device_id_type=pl.DeviceIdType.MESH,
            )
            r.start()
            p1_rdmas.append(r)

        keep_partial = masked_partial([4 * Z + q for Z in range(4)])
        x_mine = x_ref[pl.ds(p * CHUNK, CHUNK), :]
        shared_mine = jnp.dot(x_mine, sw_ref[:, :],
                              preferred_element_type=jnp.float32)

        for r in p1_rdmas:
            r.wait_recv()
        acc_ref[:, :] = (keep_partial
                         + stage1[pl.ds(0, QROWS), :]
                         + stage1[pl.ds(QROWS, QROWS), :]
                         + stage1[pl.ds(2 * QROWS, QROWS), :])

        p2_rdmas = []
        for k in range(1, 4):
            r = pltpu.make_async_remote_copy(
                src_ref=acc_ref.at[pl.ds((z ^ k) * CHUNK, CHUNK), :],
                dst_ref=stage2.at[pl.ds((k - 1) * CHUNK, CHUNK), :],
                send_sem=send_sems.at[3 + k - 1],
                recv_sem=recv_sems.at[3 + k - 1],
                device_id=(col_partners[k - 1],),
                device_id_type=pl.DeviceIdType.MESH,
            )
            r.start()
            p2_rdmas.append(r)
        for r in p2_rdmas:
            r.wait_recv()

        out_ref[:, :] = (acc_ref[pl.ds(z * CHUNK, CHUNK), :]
                         + stage2[pl.ds(0, CHUNK), :]
                         + stage2[pl.ds(CHUNK, CHUNK), :]
                         + stage2[pl.ds(2 * CHUNK, CHUNK), :]
                         + shared_mine)

        for r in p1_rdmas + p2_rdmas:
            r.wait_send()

    return pl.pallas_call(
        body,
        out_shape=jax.ShapeDtypeStruct((CHUNK, D_OUT), jnp.float32),
        in_specs=[
            pl.BlockSpec(memory_space=pltpu.VMEM),
            pl.BlockSpec(memory_space=pltpu.VMEM),
            pl.BlockSpec(memory_space=pltpu.VMEM),
            pl.BlockSpec(memory_space=pltpu.VMEM),
            pl.BlockSpec(memory_space=pltpu.VMEM),
        ],
        out_specs=pl.BlockSpec(memory_space=pltpu.VMEM),
        scratch_shapes=[
            pltpu.VMEM((QROWS, D_OUT), jnp.float32),
            pltpu.VMEM((3 * QROWS, D_OUT), jnp.float32),
            pltpu.VMEM((3 * QROWS, D_OUT), jnp.float32),
            pltpu.VMEM((3 * CHUNK, D_OUT), jnp.float32),
            pltpu.SemaphoreType.DMA((6,)),
            pltpu.SemaphoreType.DMA((6,)),
        ],
        compiler_params=pltpu.CompilerParams(collective_id=0),
    )(x, router_W, route_idx, expert_W, shared_W)


# device time: 22028 ns/iter; 1.0464x vs baseline; 1.0464x over previous
import jax
import jax.numpy as jnp
from jax import lax
from jax.experimental import pallas as pl
from jax.experimental.pallas import tpu as pltpu

N_DEV = 16
N_TOK = 512
N_EXP = 32
D_OUT = 512
CHUNK = N_TOK // N_DEV
QROWS = 4 * CHUNK


def kernel(x, router_W, route_idx, expert_W, shared_W):
    def body(x_ref, rw_ref, idx_ref, ew_ref, sw_ref, out_ref,
             acc_ref, send_ref, stage1, stage2, send_sems, recv_sems):
        p = lax.axis_index("i")
        z = p // 4
        q = p % 4
        qx = q ^ 1
        qy = 3 - q
        qd = 3 - qx

        plane_partners = [4 * z + qd, 4 * z + qx, 4 * z + qy]
        col_partners = [4 * (z ^ 1) + q, 4 * (z ^ 2) + q, 4 * (z ^ 3) + q]

        barrier_sem = pltpu.get_barrier_semaphore()
        for nbr in plane_partners + col_partners:
            pl.semaphore_signal(barrier_sem, inc=1, device_id=(nbr,),
                                device_id_type=pl.DeviceIdType.MESH)
        pl.semaphore_wait(barrier_sem, 6)

        e0 = 2 * p

        def masked_partial(chunks):
            xs = jnp.concatenate(
                [x_ref[pl.ds(c * CHUNK, CHUNK), :] for c in chunks], axis=0)
            idxs = jnp.concatenate(
                [idx_ref[pl.ds(c * CHUNK, CHUNK), :] for c in chunks], axis=0)
            n = CHUNK * len(chunks)
            scores = jnp.dot(xs, rw_ref[:, :],
                             preferred_element_type=jnp.float32)
            s_max = jnp.max(scores, axis=1, keepdims=True)
            ex = jnp.exp(scores - s_max)
            probs = ex / jnp.sum(ex, axis=1, keepdims=True)
            onehot = lax.broadcasted_iota(jnp.int32, (n, N_EXP), 1) == idxs
            p_sel = jnp.sum(jnp.where(onehot, probs, 0.0), axis=1,
                            keepdims=True)
            y0 = jnp.dot(xs, ew_ref[0], preferred_element_type=jnp.float32)
            y1 = jnp.dot(xs, ew_ref[1], preferred_element_type=jnp.float32)
            c0 = jnp.where(idxs == e0, p_sel, 0.0)
            c1 = jnp.where(idxs == e0 + 1, p_sel, 0.0)
            return c0 * y0 + c1 * y1

        p1_rdmas = []
        for j, qq in enumerate([qd, qx, qy]):
            rows = pl.ds(j * QROWS, QROWS)
            send_ref[rows, :] = masked_partial([4 * Z + qq for Z in range(4)])
            r = pltpu.make_async_remote_copy(
                src_ref=send_ref.at[rows, :],
                dst_ref=stage1.at[rows, :],
                send_sem=send_sems.at[j], recv_sem=recv_sems.at[j],
                device_id=(plane_partners[j],),
                device_id_type=pl.DeviceIdType.MESH,
            )
            r.start()
            p1_rdmas.append(r)

        keep_partial = masked_partial([4 * Z + q for Z in range(4)])
        x_mine = x_ref[pl.ds(p * CHUNK, CHUNK), :]
        shared_mine = jnp.dot(x_mine, sw_ref[:, :],
                              preferred_element_type=jnp.float32)

        for r in p1_rdmas:
            r.wait_recv()

        acc_ref[:, :] = keep_partial
        p2_rdmas = []
        for k in [1, 2, 3, 0]:
            blk = pl.ds((z ^ k) * CHUNK, CHUNK)
            off = (z ^ k) * CHUNK
            acc_ref[blk, :] = (
                acc_ref[blk, :]
                + stage1[pl.ds(off, CHUNK), :]
                + stage1[pl.ds(QROWS + off, CHUNK), :]
                + stage1[pl.ds(2 * QROWS + off, CHUNK), :])
            if k == 0:
                continue
            r = pltpu.make_async_remote_copy(
                src_ref=acc_ref.at[blk, :],
                dst_ref=stage2.at[pl.ds((k - 1) * CHUNK, CHUNK), :],
                send_sem=send_sems.at[3 + k - 1],
                recv_sem=recv_sems.at[3 + k - 1],
                device_id=(col_partners[k - 1],),
                device_id_type=pl.DeviceIdType.MESH,
            )
            r.start()
            p2_rdmas.append(r)
        for r in p2_rdmas:
            r.wait_recv()

        out_ref[:, :] = (acc_ref[pl.ds(z * CHUNK, CHUNK), :]
                         + stage2[pl.ds(0, CHUNK), :]
                         + stage2[pl.ds(CHUNK, CHUNK), :]
                         + stage2[pl.ds(2 * CHUNK, CHUNK), :]
                         + shared_mine)

        for r in p1_rdmas + p2_rdmas:
            r.wait_send()

    return pl.pallas_call(
        body,
        out_shape=jax.ShapeDtypeStruct((CHUNK, D_OUT), jnp.float32),
        in_specs=[
            pl.BlockSpec(memory_space=pltpu.VMEM),
            pl.BlockSpec(memory_space=pltpu.VMEM),
            pl.BlockSpec(memory_space=pltpu.VMEM),
            pl.BlockSpec(memory_space=pltpu.VMEM),
            pl.BlockSpec(memory_space=pltpu.VMEM),
        ],
        out_specs=pl.BlockSpec(memory_space=pltpu.VMEM),
        scratch_shapes=[
            pltpu.VMEM((QROWS, D_OUT), jnp.float32),
            pltpu.VMEM((3 * QROWS, D_OUT), jnp.float32),
            pltpu.VMEM((3 * QROWS, D_OUT), jnp.float32),
            pltpu.VMEM((3 * CHUNK, D_OUT), jnp.float32),
            pltpu.SemaphoreType.DMA((6,)),
            pltpu.SemaphoreType.DMA((6,)),
        ],
        compiler_params=pltpu.CompilerParams(collective_id=0),
    )(x, router_W, route_idx, expert_W, shared_W)


# device time: 20713 ns/iter; 1.1128x vs baseline; 1.0635x over previous
import jax
import jax.numpy as jnp
from jax import lax
from jax.experimental import pallas as pl
from jax.experimental.pallas import tpu as pltpu

N_DEV = 16
N_TOK = 512
N_EXP = 32
D_OUT = 512
CHUNK = N_TOK // N_DEV
QROWS = 4 * CHUNK


def kernel(x, router_W, route_idx, expert_W, shared_W):
    def body(x_ref, rw_ref, idx_ref, ew_ref, sw_ref, out_ref,
             acc_ref, send_ref, stage1, stage2, send_sems, recv_sems):
        p = lax.axis_index("i")
        z = p // 4
        q = p % 4
        qx = q ^ 1
        qy = 3 - q
        qd = 3 - qx

        plane_partners = [4 * z + qd, 4 * z + qx, 4 * z + qy]
        col_partners = [4 * (z ^ 1) + q, 4 * (z ^ 2) + q, 4 * (z ^ 3) + q]

        barrier_sem = pltpu.get_barrier_semaphore()
        for nbr in plane_partners + col_partners:
            pl.semaphore_signal(barrier_sem, inc=1, device_id=(nbr,),
                                device_id_type=pl.DeviceIdType.MESH)
        pl.semaphore_wait(barrier_sem, 6)

        e0 = 2 * p

        def masked_partial(chunks):
            xs = jnp.concatenate(
                [x_ref[pl.ds(c * CHUNK, CHUNK), :] for c in chunks], axis=0)
            idxs = jnp.concatenate(
                [idx_ref[pl.ds(c * CHUNK, CHUNK), :] for c in chunks], axis=0)
            n = CHUNK * len(chunks)
            scores = jnp.dot(xs, rw_ref[:, :],
                             preferred_element_type=jnp.float32)
            s_max = jnp.max(scores, axis=1, keepdims=True)
            ex = jnp.exp(scores - s_max)
            probs = ex / jnp.sum(ex, axis=1, keepdims=True)
            onehot = lax.broadcasted_iota(jnp.int32, (n, N_EXP), 1) == idxs
            p_sel = jnp.sum(jnp.where(onehot, probs, 0.0), axis=1,
                            keepdims=True)
            y0 = jnp.dot(xs, ew_ref[0], preferred_element_type=jnp.float32)
            y1 = jnp.dot(xs, ew_ref[1], preferred_element_type=jnp.float32)
            c0 = jnp.where(idxs == e0, p_sel, 0.0)
            c1 = jnp.where(idxs == e0 + 1, p_sel, 0.0)
            return c0 * y0 + c1 * y1

        p1_rdmas = []
        for j, qq in enumerate([qd, qx, qy]):
            rows = pl.ds(j * QROWS, QROWS)
            send_ref[rows, :] = masked_partial([4 * Z + qq for Z in range(4)])
            r = pltpu.make_async_remote_copy(
                src_ref=send_ref.at[rows, :],
                dst_ref=stage1.at[rows, :],
                send_sem=send_sems.at[j], recv_sem=recv_sems.at[j],
                device_id=(plane_partners[j],),
                device_id_type=pl.DeviceIdType.MESH,
            )
            r.start()
            p1_rdmas.append(r)

        keep_partial = masked_partial([4 * Z + q for Z in range(4)])
        x_mine = x_ref[pl.ds(p * CHUNK, CHUNK), :]
        shared_mine = jnp.dot(x_mine, sw_ref[:, :],
                              preferred_element_type=jnp.float32)

        for r in p1_rdmas:
            r.wait_recv()

        p2_rdmas = []
        for Z in range(4):
            blk = pl.ds(Z * CHUNK, CHUNK)
            acc_ref[blk, :] = (
                keep_partial[Z * CHUNK:(Z + 1) * CHUNK, :]
                + stage1[pl.ds(Z * CHUNK, CHUNK), :]
                + stage1[pl.ds(QROWS + Z * CHUNK, CHUNK), :]
                + stage1[pl.ds(2 * QROWS + Z * CHUNK, CHUNK), :])
            k = Z ^ z
            r = pltpu.make_async_remote_copy(
                src_ref=acc_ref.at[blk, :],
                dst_ref=stage2.at[pl.ds(jnp.maximum(k - 1, 0) * CHUNK, CHUNK), :],
                send_sem=send_sems.at[3],
                recv_sem=recv_sems.at[3],
                device_id=(4 * Z + q,),
                device_id_type=pl.DeviceIdType.MESH,
            )

            @pl.when(k != 0)
            def _():
                r.start()

            p2_rdmas.append((r, k))
        for r, k in p2_rdmas:
            @pl.when(k != 0)
            def _():
                r.wait_recv()

        out_ref[:, :] = (acc_ref[pl.ds(z * CHUNK, CHUNK), :]
                         + stage2[pl.ds(0, CHUNK), :]
                         + stage2[pl.ds(CHUNK, CHUNK), :]
                         + stage2[pl.ds(2 * CHUNK, CHUNK), :]
                         + shared_mine)

        for r in p1_rdmas:
            r.wait_send()
        for r, k in p2_rdmas:
            @pl.when(k != 0)
            def _():
                r.wait_send()

    return pl.pallas_call(
        body,
        out_shape=jax.ShapeDtypeStruct((CHUNK, D_OUT), jnp.float32),
        in_specs=[
            pl.BlockSpec(memory_space=pltpu.VMEM),
            pl.BlockSpec(memory_space=pltpu.VMEM),
            pl.BlockSpec(memory_space=pltpu.VMEM),
            pl.BlockSpec(memory_space=pltpu.VMEM),
            pl.BlockSpec(memory_space=pltpu.VMEM),
        ],
        out_specs=pl.BlockSpec(memory_space=pltpu.VMEM),
        scratch_shapes=[
            pltpu.VMEM((QROWS, D_OUT), jnp.float32),
            pltpu.VMEM((3 * QROWS, D_OUT), jnp.float32),
            pltpu.VMEM((3 * QROWS, D_OUT), jnp.float32),
            pltpu.VMEM((3 * CHUNK, D_OUT), jnp.float32),
            pltpu.SemaphoreType.DMA((6,)),
            pltpu.SemaphoreType.DMA((6,)),
        ],
        compiler_params=pltpu.CompilerParams(collective_id=0),
    )(x, router_W, route_idx, expert_W, shared_W)
